# Initial kernel scaffold; baseline (speedup 1.0000x reference)
import functools

import jax
import jax.numpy as jnp
from jax import lax
from jax.experimental import pallas as pl
from jax.experimental.pallas import tpu as pltpu
from jax.experimental.pallas import tpu_sc as plsc

N = 100000
E = 3200000
HID1 = 40
G1 = 24
HID2 = 24
ADIM = 8

NC = 2
NS = 16
NPAD = 100352
SLAB = NPAD // NS

E_PER_TILE = E // NS
CA = 4000
NCHUNK_A = E_PER_TILE // CA
CB = 2000
NCHUNK_B = E_PER_TILE // CB

BM = 800
GRID = N // BM

_mesh = plsc.VectorSubcoreMesh(core_axis_name="c", subcore_axis_name="s")


@functools.partial(
    pl.kernel,
    out_type=jax.ShapeDtypeStruct((NC, NPAD), jnp.float32),
    mesh=_mesh,
    scratch_types=[
        pltpu.VMEM((CA,), jnp.int32),
        pltpu.VMEM((CA,), jnp.float32),
        pltpu.VMEM((SLAB,), jnp.float32),
        pltpu.VMEM_SHARED((NPAD,), jnp.float32),
    ],
)
def _degrees(ei, deg_out, idxb, onesb, zb, deg_sh):
    c = lax.axis_index("c")
    s = lax.axis_index("s")
    one16 = jnp.full((16,), 1.0, jnp.float32)
    zero16 = jnp.zeros((16,), jnp.float32)

    def _ones(i, _):
        onesb[pl.ds(i * 16, 16)] = one16
        return 0

    lax.fori_loop(0, CA // 16, _ones, 0)

    def _zeros(i, _):
        zb[pl.ds(i * 16, 16)] = zero16
        return 0

    lax.fori_loop(0, SLAB // 16, _zeros, 0)
    pltpu.sync_copy(zb, deg_sh.at[pl.ds(s * SLAB, SLAB)])
    plsc.subcore_barrier()

    base = s * E_PER_TILE

    def _chunk(k, _):
        pltpu.sync_copy(ei.at[c, pl.ds(base + k * CA, CA)], idxb)
        pltpu.sync_copy(onesb, deg_sh.at[idxb], add=True)
        return 0

    lax.fori_loop(0, NCHUNK_A, _chunk, 0)
    plsc.subcore_barrier()
    pltpu.sync_copy(deg_sh.at[pl.ds(s * SLAB, SLAB)],
                    deg_out.at[c, pl.ds(s * SLAB, SLAB)])


@functools.partial(
    pl.kernel,
    out_type=(
        jax.ShapeDtypeStruct((NC, NPAD, 16), jnp.float32),
        jax.ShapeDtypeStruct((NC, NPAD), jnp.float32),
    ),
    mesh=_mesh,
    scratch_types=[
        pltpu.VMEM((CB,), jnp.int32),
        pltpu.VMEM((CB,), jnp.int32),
        pltpu.VMEM((CB, 16), jnp.float32),
        pltpu.VMEM((CB,), jnp.float32),
        pltpu.VMEM((SLAB,), jnp.float32),
        pltpu.VMEM((1568, 16), jnp.float32),
        pltpu.VMEM_SHARED((NPAD, 16), jnp.float32),
        pltpu.VMEM_SHARED((NPAD,), jnp.float32),
        pltpu.VMEM_SHARED((NPAD,), jnp.float32),
        pltpu.SemaphoreType.DMA,
    ],
)
def _conv_scatter(ei, hw, innorm, agg_out, w_out,
                  srcb, dstb, rows, vals, zb, zrows,
                  agg_sh, innorm_sh, w_sh, sem):
    c = lax.axis_index("c")
    s = lax.axis_index("s")
    zero16 = jnp.zeros((16,), jnp.float32)

    pltpu.sync_copy(innorm.at[pl.ds(s * SLAB, SLAB)],
                    innorm_sh.at[pl.ds(s * SLAB, SLAB)])

    def _zeros(i, _):
        zb[pl.ds(i * 16, 16)] = zero16
        return 0

    lax.fori_loop(0, SLAB // 16, _zeros, 0)
    pltpu.sync_copy(zb, w_sh.at[pl.ds(s * SLAB, SLAB)])

    def _zrows(i, _):
        zrows[i] = zero16
        return 0

    lax.fori_loop(0, 1568, _zrows, 0)
    for j in range(4):
        pltpu.sync_copy(zrows, agg_sh.at[pl.ds(s * SLAB + j * 1568, 1568)])
    plsc.subcore_barrier()

    base = s * E_PER_TILE
    half = NCHUNK_B // 2

    def _chunk(k, _):
        pltpu.sync_copy(ei.at[0, pl.ds(base + k * CB, CB)], srcb)
        pltpu.sync_copy(ei.at[1, pl.ds(base + k * CB, CB)], dstb)
        pltpu.async_copy(hw.at[c].at[srcb], rows, sem).wait()
        pltpu.sync_copy(rows, agg_sh.at[dstb], add=True)

        @pl.when((k >= c * half) & (k < (c + 1) * half))
        def _():
            pltpu.async_copy(innorm_sh.at[dstb], vals, sem).wait()
            pltpu.sync_copy(vals, w_sh.at[srcb], add=True)

        return 0

    lax.fori_loop(0, NCHUNK_B, _chunk, 0)
    plsc.subcore_barrier()
    pltpu.sync_copy(agg_sh.at[pl.ds(s * SLAB, SLAB)],
                    agg_out.at[c, pl.ds(s * SLAB, SLAB)])
    pltpu.sync_copy(w_sh.at[pl.ds(s * SLAB, SLAB)],
                    w_out.at[c, pl.ds(s * SLAB, SLAB)])


def _fc1_body(nd_ref, nf_ref, w1a_ref, w1b_ref, b1_ref, h1_ref, s1_ref, s2_ref):
    i = pl.program_id(0)
    h = (jnp.dot(nd_ref[...], w1a_ref[...], preferred_element_type=jnp.float32)
         + jnp.dot(nf_ref[...], w1b_ref[...], preferred_element_type=jnp.float32)
         + b1_ref[...])
    h = jnp.maximum(h, 0.0)
    h1_ref[...] = h

    @pl.when(i == 0)
    def _():
        s1_ref[...] = jnp.zeros_like(s1_ref)
        s2_ref[...] = jnp.zeros_like(s2_ref)

    s1_ref[...] += jnp.sum(h, axis=0, keepdims=True)
    s2_ref[...] += jnp.sum(h * h, axis=0, keepdims=True)


_fc1_call = pl.pallas_call(
    _fc1_body,
    grid=(GRID,),
    in_specs=[
        pl.BlockSpec((BM, 64), lambda i: (i, 0)),
        pl.BlockSpec((BM, 64), lambda i: (i, 0)),
        pl.BlockSpec((64, HID1), lambda i: (0, 0)),
        pl.BlockSpec((64, HID1), lambda i: (0, 0)),
        pl.BlockSpec((1, HID1), lambda i: (0, 0)),
    ],
    out_specs=[
        pl.BlockSpec((BM, HID1), lambda i: (i, 0)),
        pl.BlockSpec((1, HID1), lambda i: (0, 0)),
        pl.BlockSpec((1, HID1), lambda i: (0, 0)),
    ],
    out_shape=[
        jax.ShapeDtypeStruct((N, HID1), jnp.float32),
        jax.ShapeDtypeStruct((1, HID1), jnp.float32),
        jax.ShapeDtypeStruct((1, HID1), jnp.float32),
    ],
)


def _hw_body(h1_ref, a_ref, csh_ref, on_ref, w1p_ref, hw_ref):
    hn = h1_ref[...] * a_ref[...] + csh_ref[...]
    hn = hn * on_ref[...]
    hw = jnp.dot(hn, w1p_ref[...], preferred_element_type=jnp.float32)
    hw_ref[0] = hw[:, :16]
    hw_ref[1] = hw[:, 16:]


_hw_call = pl.pallas_call(
    _hw_body,
    grid=(GRID,),
    in_specs=[
        pl.BlockSpec((BM, HID1), lambda i: (i, 0)),
        pl.BlockSpec((1, HID1), lambda i: (0, 0)),
        pl.BlockSpec((1, HID1), lambda i: (0, 0)),
        pl.BlockSpec((BM, 1), lambda i: (i, 0)),
        pl.BlockSpec((HID1, 32), lambda i: (0, 0)),
    ],
    out_specs=pl.BlockSpec((2, BM, 16), lambda i: (0, i, 0)),
    out_shape=jax.ShapeDtypeStruct((2, N, 16), jnp.float32),
)


def _final_body(agg_ref, w_ref, innorm_ref, onorm_ref, act_ref,
                wg0_ref, wg1_ref, wa_ref, b2p_ref, c2w_ref, b2c_ref,
                f3t_ref, f3b_ref, s_ref, q_ref):
    i = pl.program_id(0)
    t = (jnp.dot(agg_ref[0], wg0_ref[...], preferred_element_type=jnp.float32)
         + jnp.dot(agg_ref[1], wg1_ref[...], preferred_element_type=jnp.float32))
    z = (t * innorm_ref[...]
         + jnp.dot(act_ref[...], wa_ref[...], preferred_element_type=jnp.float32)
         + b2p_ref[...])
    h2 = jnp.maximum(z, 0.0)
    coef = (w_ref[0:1, :] + w_ref[1:2, :]) * onorm_ref[...]

    @pl.when(i == 0)
    def _():
        s_ref[...] = jnp.zeros_like(s_ref)

    s_ref[...] += jnp.dot(coef, h2, preferred_element_type=jnp.float32)

    @pl.when(i == GRID - 1)
    def _():
        sm = s_ref[...] * (1.0 / N)
        g = jnp.dot(sm, c2w_ref[...], preferred_element_type=jnp.float32) + b2c_ref[...]
        q_ref[...] = jnp.dot(g, f3t_ref[...], preferred_element_type=jnp.float32) + f3b_ref[...]


_final_call = pl.pallas_call(
    _final_body,
    grid=(GRID,),
    in_specs=[
        pl.BlockSpec((2, BM, 16), lambda i: (0, i, 0)),
        pl.BlockSpec((2, BM), lambda i: (0, i)),
        pl.BlockSpec((BM, 1), lambda i: (i, 0)),
        pl.BlockSpec((1, BM), lambda i: (0, i)),
        pl.BlockSpec((BM, ADIM), lambda i: (i, 0)),
        pl.BlockSpec((16, HID2), lambda i: (0, 0)),
        pl.BlockSpec((16, HID2), lambda i: (0, 0)),
        pl.BlockSpec((ADIM, HID2), lambda i: (0, 0)),
        pl.BlockSpec((1, HID2), lambda i: (0, 0)),
        pl.BlockSpec((HID2, G1), lambda i: (0, 0)),
        pl.BlockSpec((1, G1), lambda i: (0, 0)),
        pl.BlockSpec((G1, 1), lambda i: (0, 0)),
        pl.BlockSpec((1, 1), lambda i: (0, 0)),
    ],
    out_specs=[
        pl.BlockSpec((1, HID2), lambda i: (0, 0)),
        pl.BlockSpec((1, 1), lambda i: (0, 0)),
    ],
    out_shape=[
        jax.ShapeDtypeStruct((1, HID2), jnp.float32),
        jax.ShapeDtypeStruct((1, 1), jnp.float32),
    ],
)


def kernel(n_delay, n_feat, edge_index, action, fc1_W, fc1_b, bn_gamma,
           bn_beta, conv1_W, conv1_b, fc2_W, fc2_b, conv2_W, conv2_b,
           fc3_W, fc3_b):
    ei = edge_index.astype(jnp.int32)

    degs = _degrees(ei)
    out_norm = lax.rsqrt(jnp.clip(degs[0, :N], 1.0, None))
    in_norm_pad = lax.rsqrt(jnp.clip(degs[1], 1.0, None))

    w1t = fc1_W.T
    h1, s1, s2 = _fc1_call(n_delay, n_feat, w1t[:64], w1t[64:],
                           fc1_b.reshape(1, HID1))
    mean = s1 * (1.0 / N)
    var = s2 * (1.0 / N) - mean * mean
    a = bn_gamma.reshape(1, HID1) * lax.rsqrt(var + 1e-5)
    csh = bn_beta.reshape(1, HID1) - mean * a

    w1p = jnp.concatenate(
        [conv1_W, jnp.zeros((HID1, 32 - G1), jnp.float32)], axis=1)
    hw = _hw_call(h1, a, csh, out_norm.reshape(N, 1), w1p)

    agg, w01 = _conv_scatter(ei, hw, in_norm_pad)

    w2t = fc2_W.T
    wg = w2t[:G1]
    wg32 = jnp.concatenate([wg, jnp.zeros((32 - G1, HID2), jnp.float32)], axis=0)
    b2p = (fc2_b + conv1_b @ wg).reshape(1, HID2)
    _, q = _final_call(
        agg, w01, in_norm_pad[:N].reshape(N, 1), out_norm.reshape(1, N),
        action, wg32[:16], wg32[16:], w2t[G1:], b2p,
        conv2_W, conv2_b.reshape(1, G1), fc3_W.T, fc3_b.reshape(1, 1))
    return q.reshape(())


# speedup vs baseline: 23.2566x; 23.2566x over previous
import functools

import jax
import jax.numpy as jnp
from jax import lax
from jax.experimental import pallas as pl
from jax.experimental.pallas import tpu as pltpu
from jax.experimental.pallas import tpu_sc as plsc

N = 100000
E = 3200000
HID1 = 40
G1 = 24
HID2 = 24
ADIM = 8
CH = 16

NC = 2
NS = 16
NPAD = 100352
SLAB = NPAD // NS

CA = 1024
NCHUNK_A = E // CA
CB = 1024
NCHUNK_B = E // CB

BM = 800
GRID = N // BM

_mesh = plsc.VectorSubcoreMesh(core_axis_name="c", subcore_axis_name="s",
                               num_cores=NC, num_subcores=NS)


@functools.partial(
    pl.kernel,
    out_type=jax.ShapeDtypeStruct((NC, NPAD), jnp.float32),
    mesh=_mesh,
    scratch_types=[
        pltpu.VMEM((CA,), jnp.int32),
        pltpu.VMEM((CA,), jnp.float32),
        pltpu.VMEM((SLAB,), jnp.float32),
        pltpu.VMEM_SHARED((NPAD,), jnp.float32),
    ],
    compiler_params=pltpu.CompilerParams(use_tc_tiling_on_sc=False),
)
def _degrees(ei, deg_out, idxb, onesb, zb, deg_sh):
    c = lax.axis_index("c")
    s = lax.axis_index("s")
    one16 = jnp.full((16,), 1.0, jnp.float32)
    zero16 = jnp.zeros((16,), jnp.float32)

    def _ones(i, _):
        onesb[pl.ds(i * 16, 16)] = one16
        return 0

    lax.fori_loop(0, CA // 16, _ones, 0)

    def _zeros(i, _):
        zb[pl.ds(i * 16, 16)] = zero16
        return 0

    lax.fori_loop(0, SLAB // 16, _zeros, 0)
    pltpu.sync_copy(zb, deg_sh.at[pl.ds(s * SLAB, SLAB)])
    plsc.subcore_barrier()

    ntrips = (NCHUNK_A - s + NS - 1) // NS

    def _chunk(k, _):
        g = s + k * NS
        pltpu.sync_copy(ei.at[pl.ds(c * E + g * CA, CA)], idxb)
        pltpu.sync_copy(onesb, deg_sh.at[idxb], add=True)
        return 0

    lax.fori_loop(0, ntrips, _chunk, 0)
    plsc.subcore_barrier()
    pltpu.sync_copy(deg_sh.at[pl.ds(s * SLAB, SLAB)],
                    deg_out.at[c, pl.ds(s * SLAB, SLAB)])


@functools.partial(
    pl.kernel,
    out_type=jax.ShapeDtypeStruct((NC, NPAD, CH), jnp.float32),
    mesh=_mesh,
    scratch_types=[
        pltpu.VMEM((CB,), jnp.int32),
        pltpu.VMEM((CB,), jnp.int32),
        pltpu.VMEM((CB, CH), jnp.float32),
        pltpu.VMEM_SHARED((NPAD, CH), jnp.float32),
        pltpu.SemaphoreType.DMA,
    ],
    compiler_params=pltpu.CompilerParams(use_tc_tiling_on_sc=False),
)
def _conv_scatter(ei, hw, z2d, agg_out, srcb, dstb, rows, agg_sh, sem):
    c = lax.axis_index("c")
    s = lax.axis_index("s")

    pltpu.sync_copy(z2d.at[pl.ds(s * SLAB, SLAB)],
                    agg_sh.at[pl.ds(s * SLAB, SLAB)])
    plsc.subcore_barrier()

    ntrips = (NCHUNK_B - s + NS - 1) // NS

    def _chunk(k, _):
        g = s + k * NS
        pltpu.sync_copy(ei.at[pl.ds(g * CB, CB)], srcb)
        pltpu.sync_copy(ei.at[pl.ds(E + g * CB, CB)], dstb)
        pltpu.async_copy(hw.at[c].at[srcb], rows, sem).wait()
        pltpu.sync_copy(rows, agg_sh.at[dstb], add=True)
        return 0

    lax.fori_loop(0, ntrips, _chunk, 0)
    plsc.subcore_barrier()
    pltpu.sync_copy(agg_sh.at[pl.ds(s * SLAB, SLAB)],
                    agg_out.at[c, pl.ds(s * SLAB, SLAB)])


@functools.partial(
    pl.kernel,
    out_type=jax.ShapeDtypeStruct((NC, NPAD), jnp.float32),
    mesh=_mesh,
    scratch_types=[
        pltpu.VMEM((CB,), jnp.int32),
        pltpu.VMEM((CB,), jnp.int32),
        pltpu.VMEM((CB,), jnp.float32),
        pltpu.VMEM((SLAB,), jnp.float32),
        pltpu.VMEM_SHARED((NPAD,), jnp.float32),
        pltpu.VMEM_SHARED((NPAD,), jnp.float32),
        pltpu.SemaphoreType.DMA,
    ],
    compiler_params=pltpu.CompilerParams(use_tc_tiling_on_sc=False),
)
def _w_scatter(ei, innorm, w_out, srcb, dstb, vals, zb, innorm_sh, w_sh, sem):
    c = lax.axis_index("c")
    s = lax.axis_index("s")
    zero16 = jnp.zeros((16,), jnp.float32)

    pltpu.sync_copy(innorm.at[pl.ds(s * SLAB, SLAB)],
                    innorm_sh.at[pl.ds(s * SLAB, SLAB)])

    def _zeros(i, _):
        zb[pl.ds(i * 16, 16)] = zero16
        return 0

    lax.fori_loop(0, SLAB // 16, _zeros, 0)
    pltpu.sync_copy(zb, w_sh.at[pl.ds(s * SLAB, SLAB)])
    plsc.subcore_barrier()

    ntrips = (NCHUNK_B - s + NS - 1) // NS

    def _chunk(k, _):
        g = s + k * NS

        @pl.when(g % 2 == c)
        def _():
            pltpu.sync_copy(ei.at[pl.ds(g * CB, CB)], srcb)
            pltpu.sync_copy(ei.at[pl.ds(E + g * CB, CB)], dstb)
            pltpu.async_copy(innorm_sh.at[dstb], vals, sem).wait()
            pltpu.sync_copy(vals, w_sh.at[srcb], add=True)

        return 0

    lax.fori_loop(0, ntrips, _chunk, 0)
    plsc.subcore_barrier()
    pltpu.sync_copy(w_sh.at[pl.ds(s * SLAB, SLAB)],
                    w_out.at[c, pl.ds(s * SLAB, SLAB)])


def _fc1_body(nd_ref, nf_ref, w1a_ref, w1b_ref, b1_ref, h1_ref, s1_ref, s2_ref):
    i = pl.program_id(0)
    h = (jnp.dot(nd_ref[...], w1a_ref[...], preferred_element_type=jnp.float32)
         + jnp.dot(nf_ref[...], w1b_ref[...], preferred_element_type=jnp.float32)
         + b1_ref[...])
    h = jnp.maximum(h, 0.0)
    h1_ref[...] = h

    @pl.when(i == 0)
    def _():
        s1_ref[...] = jnp.zeros_like(s1_ref)
        s2_ref[...] = jnp.zeros_like(s2_ref)

    s1_ref[...] += jnp.sum(h, axis=0, keepdims=True)
    s2_ref[...] += jnp.sum(h * h, axis=0, keepdims=True)


_fc1_call = pl.pallas_call(
    _fc1_body,
    grid=(GRID,),
    in_specs=[
        pl.BlockSpec((BM, 64), lambda i: (i, 0)),
        pl.BlockSpec((BM, 64), lambda i: (i, 0)),
        pl.BlockSpec((64, HID1), lambda i: (0, 0)),
        pl.BlockSpec((64, HID1), lambda i: (0, 0)),
        pl.BlockSpec((1, HID1), lambda i: (0, 0)),
    ],
    out_specs=[
        pl.BlockSpec((BM, HID1), lambda i: (i, 0)),
        pl.BlockSpec((1, HID1), lambda i: (0, 0)),
        pl.BlockSpec((1, HID1), lambda i: (0, 0)),
    ],
    out_shape=[
        jax.ShapeDtypeStruct((N, HID1), jnp.float32),
        jax.ShapeDtypeStruct((1, HID1), jnp.float32),
        jax.ShapeDtypeStruct((1, HID1), jnp.float32),
    ],
)


def _hw_body(h1_ref, a_ref, csh_ref, on_ref, w1p_ref, hw_ref):
    hn = h1_ref[...] * a_ref[...] + csh_ref[...]
    hn = hn * on_ref[...]
    hw = jnp.dot(hn, w1p_ref[...], preferred_element_type=jnp.float32)
    hw_ref[0] = hw[:, :CH]
    hw_ref[1] = hw[:, CH:]


_hw_call = pl.pallas_call(
    _hw_body,
    grid=(GRID,),
    in_specs=[
        pl.BlockSpec((BM, HID1), lambda i: (i, 0)),
        pl.BlockSpec((1, HID1), lambda i: (0, 0)),
        pl.BlockSpec((1, HID1), lambda i: (0, 0)),
        pl.BlockSpec((BM, 1), lambda i: (i, 0)),
        pl.BlockSpec((HID1, 2 * CH), lambda i: (0, 0)),
    ],
    out_specs=pl.BlockSpec((2, BM, CH), lambda i: (0, i, 0)),
    out_shape=jax.ShapeDtypeStruct((2, N, CH), jnp.float32),
)


def _final_body(agg_ref, w_ref, innorm_ref, onorm_ref, act_ref,
                wg0_ref, wg1_ref, wa_ref, b2p_ref, c2w_ref, b2c_ref,
                f3t_ref, f3b_ref, s_ref, q_ref):
    i = pl.program_id(0)
    t = (jnp.dot(agg_ref[0], wg0_ref[...], preferred_element_type=jnp.float32)
         + jnp.dot(agg_ref[1], wg1_ref[...], preferred_element_type=jnp.float32))
    z = (t * innorm_ref[...]
         + jnp.dot(act_ref[...], wa_ref[...], preferred_element_type=jnp.float32)
         + b2p_ref[...])
    h2 = jnp.maximum(z, 0.0)
    coef = (w_ref[:, 0:1] + w_ref[:, 1:2]) * onorm_ref[...]

    @pl.when(i == 0)
    def _():
        s_ref[...] = jnp.zeros_like(s_ref)

    s_ref[...] += jnp.sum(coef * h2, axis=0, keepdims=True)

    @pl.when(i == GRID - 1)
    def _():
        sm = s_ref[...] * (1.0 / N)
        g = jnp.dot(sm, c2w_ref[...], preferred_element_type=jnp.float32) + b2c_ref[...]
        q_ref[...] = jnp.dot(g, f3t_ref[...], preferred_element_type=jnp.float32) + f3b_ref[...]


_final_call = pl.pallas_call(
    _final_body,
    grid=(GRID,),
    in_specs=[
        pl.BlockSpec((2, BM, CH), lambda i: (0, i, 0)),
        pl.BlockSpec((BM, 2), lambda i: (i, 0)),
        pl.BlockSpec((BM, 1), lambda i: (i, 0)),
        pl.BlockSpec((BM, 1), lambda i: (i, 0)),
        pl.BlockSpec((BM, ADIM), lambda i: (i, 0)),
        pl.BlockSpec((CH, HID2), lambda i: (0, 0)),
        pl.BlockSpec((CH, HID2), lambda i: (0, 0)),
        pl.BlockSpec((ADIM, HID2), lambda i: (0, 0)),
        pl.BlockSpec((1, HID2), lambda i: (0, 0)),
        pl.BlockSpec((HID2, G1), lambda i: (0, 0)),
        pl.BlockSpec((1, G1), lambda i: (0, 0)),
        pl.BlockSpec((G1, 1), lambda i: (0, 0)),
        pl.BlockSpec((1, 1), lambda i: (0, 0)),
    ],
    out_specs=[
        pl.BlockSpec((1, HID2), lambda i: (0, 0)),
        pl.BlockSpec((1, 1), lambda i: (0, 0)),
    ],
    out_shape=[
        jax.ShapeDtypeStruct((1, HID2), jnp.float32),
        jax.ShapeDtypeStruct((1, 1), jnp.float32),
    ],
)


def kernel(n_delay, n_feat, edge_index, action, fc1_W, fc1_b, bn_gamma,
           bn_beta, conv1_W, conv1_b, fc2_W, fc2_b, conv2_W, conv2_b,
           fc3_W, fc3_b):
    ei = edge_index.astype(jnp.int32).reshape(2 * E)

    degs = _degrees(ei)
    out_norm = lax.rsqrt(jnp.clip(degs[0, :N], 1.0, None))
    in_norm_pad = lax.rsqrt(jnp.clip(degs[1], 1.0, None))

    w1t = fc1_W.T
    h1, s1, s2 = _fc1_call(n_delay, n_feat, w1t[:64], w1t[64:],
                           fc1_b.reshape(1, HID1))
    mean = s1 * (1.0 / N)
    var = s2 * (1.0 / N) - mean * mean
    a = bn_gamma.reshape(1, HID1) * lax.rsqrt(var + 1e-5)
    csh = bn_beta.reshape(1, HID1) - mean * a

    w1p = jnp.concatenate(
        [conv1_W, jnp.zeros((HID1, 2 * CH - G1), jnp.float32)], axis=1)
    hw = _hw_call(h1, a, csh, out_norm.reshape(N, 1), w1p)

    zeros2d = jnp.zeros((NPAD, CH), jnp.float32)
    agg = _conv_scatter(ei, hw, zeros2d)
    w01 = _w_scatter(ei, in_norm_pad)

    w2t = fc2_W.T
    wg = w2t[:G1]
    wg32 = jnp.concatenate(
        [wg, jnp.zeros((2 * CH - G1, HID2), jnp.float32)], axis=0)
    b2p = (fc2_b + conv1_b @ wg).reshape(1, HID2)
    _, q = _final_call(
        agg, w01[:, :N].T, in_norm_pad[:N].reshape(N, 1), out_norm.reshape(N, 1),
        action, wg32[:CH], wg32[CH:], w2t[G1:], b2p,
        conv2_W, conv2_b.reshape(1, G1), fc3_W.T, fc3_b.reshape(1, 1))
    return q.reshape(())


# speedup vs baseline: 24.9782x; 1.0740x over previous
import functools

import jax
import jax.numpy as jnp
from jax import lax
from jax.experimental import pallas as pl
from jax.experimental.pallas import tpu as pltpu
from jax.experimental.pallas import tpu_sc as plsc

N = 100000
E = 3200000
HID1 = 40
G1 = 24
HID2 = 24
ADIM = 8
CH = 16

NC = 2
NS = 16
NPAD = 100352
SLAB = NPAD // NS

CA = 1024
NCHUNK_A = E // CA
CB = 512
NCHUNK_B = E // CB
CW = 1024
NCHUNK_W = E // CW

BM = 800
GRID = N // BM

_mesh = plsc.VectorSubcoreMesh(core_axis_name="c", subcore_axis_name="s",
                               num_cores=NC, num_subcores=NS)


@functools.partial(
    pl.kernel,
    out_type=jax.ShapeDtypeStruct((NC, NPAD), jnp.float32),
    mesh=_mesh,
    scratch_types=[
        pltpu.VMEM((CA,), jnp.int32),
        pltpu.VMEM((CA,), jnp.float32),
        pltpu.VMEM((SLAB,), jnp.float32),
        pltpu.VMEM_SHARED((NPAD,), jnp.float32),
    ],
    compiler_params=pltpu.CompilerParams(use_tc_tiling_on_sc=False),
)
def _degrees(ei, deg_out, idxb, onesb, zb, deg_sh):
    c = lax.axis_index("c")
    s = lax.axis_index("s")
    one16 = jnp.full((16,), 1.0, jnp.float32)
    zero16 = jnp.zeros((16,), jnp.float32)

    def _ones(i, _):
        onesb[pl.ds(i * 16, 16)] = one16
        return 0

    lax.fori_loop(0, CA // 16, _ones, 0)

    def _zeros(i, _):
        zb[pl.ds(i * 16, 16)] = zero16
        return 0

    lax.fori_loop(0, SLAB // 16, _zeros, 0)
    pltpu.sync_copy(zb, deg_sh.at[pl.ds(s * SLAB, SLAB)])
    plsc.subcore_barrier()

    ntrips = (NCHUNK_A - s + NS - 1) // NS

    def _chunk(k, _):
        g = s + k * NS
        pltpu.sync_copy(ei.at[pl.ds(c * E + g * CA, CA)], idxb)
        pltpu.sync_copy(onesb, deg_sh.at[idxb], add=True)
        return 0

    lax.fori_loop(0, ntrips, _chunk, 0)
    plsc.subcore_barrier()
    pltpu.sync_copy(deg_sh.at[pl.ds(s * SLAB, SLAB)],
                    deg_out.at[c, pl.ds(s * SLAB, SLAB)])


@functools.partial(
    pl.kernel,
    out_type=jax.ShapeDtypeStruct((NC, NPAD, CH), jnp.float32),
    mesh=_mesh,
    scratch_types=[
        pltpu.VMEM((CB,), jnp.int32),
        pltpu.VMEM((CB,), jnp.int32),
        pltpu.VMEM((CB,), jnp.int32),
        pltpu.VMEM((CB,), jnp.int32),
        pltpu.VMEM((CB, CH), jnp.float32),
        pltpu.VMEM((CB, CH), jnp.float32),
        pltpu.VMEM_SHARED((NPAD, CH), jnp.float32),
        pltpu.SemaphoreType.DMA,
        pltpu.SemaphoreType.DMA,
    ],
    compiler_params=pltpu.CompilerParams(use_tc_tiling_on_sc=False),
)
def _conv_scatter(ei, hw, z2d, agg_out, srcba, srcbb, dstba, dstbb,
                  rowsa, rowsb, agg_sh, sema, semb):
    c = lax.axis_index("c")
    s = lax.axis_index("s")

    pltpu.sync_copy(z2d.at[pl.ds(s * SLAB, SLAB)],
                    agg_sh.at[pl.ds(s * SLAB, SLAB)])
    plsc.subcore_barrier()

    ntrips = (NCHUNK_B - s + NS - 1) // NS

    def _issue(k, srcb, dstb, rows, sem):
        g = s + k * NS
        pltpu.sync_copy(ei.at[pl.ds(g * CB, CB)], srcb)
        pltpu.sync_copy(ei.at[pl.ds(E + g * CB, CB)], dstb)
        pltpu.async_copy(hw.at[c].at[srcb], rows, sem)

    def _drain(srcb, dstb, rows, sem):
        pltpu.make_async_copy(hw.at[c].at[srcb], rows, sem).wait()
        pltpu.sync_copy(rows, agg_sh.at[dstb], add=True)

    _issue(0, srcba, dstba, rowsa, sema)

    def _chunk(k, _):
        @pl.when(k % 2 == 0)
        def _():
            @pl.when(k + 1 < ntrips)
            def _():
                _issue(k + 1, srcbb, dstbb, rowsb, semb)
            _drain(srcba, dstba, rowsa, sema)

        @pl.when(k % 2 == 1)
        def _():
            @pl.when(k + 1 < ntrips)
            def _():
                _issue(k + 1, srcba, dstba, rowsa, sema)
            _drain(srcbb, dstbb, rowsb, semb)

        return 0

    lax.fori_loop(0, ntrips, _chunk, 0)
    plsc.subcore_barrier()
    pltpu.sync_copy(agg_sh.at[pl.ds(s * SLAB, SLAB)],
                    agg_out.at[c, pl.ds(s * SLAB, SLAB)])


@functools.partial(
    pl.kernel,
    out_type=jax.ShapeDtypeStruct((NC, NPAD), jnp.float32),
    mesh=_mesh,
    scratch_types=[
        pltpu.VMEM((CW,), jnp.int32),
        pltpu.VMEM((CW,), jnp.int32),
        pltpu.VMEM((CW,), jnp.float32),
        pltpu.VMEM((SLAB,), jnp.float32),
        pltpu.VMEM_SHARED((NPAD,), jnp.float32),
        pltpu.VMEM_SHARED((NPAD,), jnp.float32),
        pltpu.SemaphoreType.DMA,
    ],
    compiler_params=pltpu.CompilerParams(use_tc_tiling_on_sc=False),
)
def _w_scatter(ei, innorm, w_out, srcb, dstb, vals, zb, innorm_sh, w_sh, sem):
    c = lax.axis_index("c")
    s = lax.axis_index("s")
    zero16 = jnp.zeros((16,), jnp.float32)

    pltpu.sync_copy(innorm.at[pl.ds(s * SLAB, SLAB)],
                    innorm_sh.at[pl.ds(s * SLAB, SLAB)])

    def _zeros(i, _):
        zb[pl.ds(i * 16, 16)] = zero16
        return 0

    lax.fori_loop(0, SLAB // 16, _zeros, 0)
    pltpu.sync_copy(zb, w_sh.at[pl.ds(s * SLAB, SLAB)])
    plsc.subcore_barrier()

    ntrips = (NCHUNK_W - s + NS - 1) // NS

    def _chunk(k, _):
        g = s + k * NS

        @pl.when(g % 2 == c)
        def _():
            pltpu.sync_copy(ei.at[pl.ds(g * CW, CW)], srcb)
            pltpu.sync_copy(ei.at[pl.ds(E + g * CW, CW)], dstb)
            pltpu.async_copy(innorm_sh.at[dstb], vals, sem).wait()
            pltpu.sync_copy(vals, w_sh.at[srcb], add=True)

        return 0

    lax.fori_loop(0, ntrips, _chunk, 0)
    plsc.subcore_barrier()
    pltpu.sync_copy(w_sh.at[pl.ds(s * SLAB, SLAB)],
                    w_out.at[c, pl.ds(s * SLAB, SLAB)])


def _fc1_body(nd_ref, nf_ref, w1a_ref, w1b_ref, b1_ref, h1_ref, s1_ref, s2_ref):
    i = pl.program_id(0)
    h = (jnp.dot(nd_ref[...], w1a_ref[...], preferred_element_type=jnp.float32)
         + jnp.dot(nf_ref[...], w1b_ref[...], preferred_element_type=jnp.float32)
         + b1_ref[...])
    h = jnp.maximum(h, 0.0)
    h1_ref[...] = h

    @pl.when(i == 0)
    def _():
        s1_ref[...] = jnp.zeros_like(s1_ref)
        s2_ref[...] = jnp.zeros_like(s2_ref)

    s1_ref[...] += jnp.sum(h, axis=0, keepdims=True)
    s2_ref[...] += jnp.sum(h * h, axis=0, keepdims=True)


_fc1_call = pl.pallas_call(
    _fc1_body,
    grid=(GRID,),
    in_specs=[
        pl.BlockSpec((BM, 64), lambda i: (i, 0)),
        pl.BlockSpec((BM, 64), lambda i: (i, 0)),
        pl.BlockSpec((64, HID1), lambda i: (0, 0)),
        pl.BlockSpec((64, HID1), lambda i: (0, 0)),
        pl.BlockSpec((1, HID1), lambda i: (0, 0)),
    ],
    out_specs=[
        pl.BlockSpec((BM, HID1), lambda i: (i, 0)),
        pl.BlockSpec((1, HID1), lambda i: (0, 0)),
        pl.BlockSpec((1, HID1), lambda i: (0, 0)),
    ],
    out_shape=[
        jax.ShapeDtypeStruct((N, HID1), jnp.float32),
        jax.ShapeDtypeStruct((1, HID1), jnp.float32),
        jax.ShapeDtypeStruct((1, HID1), jnp.float32),
    ],
)


def _hw_body(h1_ref, a_ref, csh_ref, on_ref, w1p_ref, hw_ref):
    hn = h1_ref[...] * a_ref[...] + csh_ref[...]
    hn = hn * on_ref[...]
    hw = jnp.dot(hn, w1p_ref[...], preferred_element_type=jnp.float32)
    hw_ref[0] = hw[:, :CH]
    hw_ref[1] = hw[:, CH:]


_hw_call = pl.pallas_call(
    _hw_body,
    grid=(GRID,),
    in_specs=[
        pl.BlockSpec((BM, HID1), lambda i: (i, 0)),
        pl.BlockSpec((1, HID1), lambda i: (0, 0)),
        pl.BlockSpec((1, HID1), lambda i: (0, 0)),
        pl.BlockSpec((BM, 1), lambda i: (i, 0)),
        pl.BlockSpec((HID1, 2 * CH), lambda i: (0, 0)),
    ],
    out_specs=pl.BlockSpec((2, BM, CH), lambda i: (0, i, 0)),
    out_shape=jax.ShapeDtypeStruct((2, N, CH), jnp.float32),
)


def _final_body(agg_ref, w_ref, innorm_ref, onorm_ref, act_ref,
                wg0_ref, wg1_ref, wa_ref, b2p_ref, c2w_ref, b2c_ref,
                f3t_ref, f3b_ref, s_ref, q_ref):
    i = pl.program_id(0)
    t = (jnp.dot(agg_ref[0], wg0_ref[...], preferred_element_type=jnp.float32)
         + jnp.dot(agg_ref[1], wg1_ref[...], preferred_element_type=jnp.float32))
    z = (t * innorm_ref[...]
         + jnp.dot(act_ref[...], wa_ref[...], preferred_element_type=jnp.float32)
         + b2p_ref[...])
    h2 = jnp.maximum(z, 0.0)
    coef = (w_ref[:, 0:1] + w_ref[:, 1:2]) * onorm_ref[...]

    @pl.when(i == 0)
    def _():
        s_ref[...] = jnp.zeros_like(s_ref)

    s_ref[...] += jnp.sum(coef * h2, axis=0, keepdims=True)

    @pl.when(i == GRID - 1)
    def _():
        sm = s_ref[...] * (1.0 / N)
        g = jnp.dot(sm, c2w_ref[...], preferred_element_type=jnp.float32) + b2c_ref[...]
        q_ref[...] = jnp.dot(g, f3t_ref[...], preferred_element_type=jnp.float32) + f3b_ref[...]


_final_call = pl.pallas_call(
    _final_body,
    grid=(GRID,),
    in_specs=[
        pl.BlockSpec((2, BM, CH), lambda i: (0, i, 0)),
        pl.BlockSpec((BM, 2), lambda i: (i, 0)),
        pl.BlockSpec((BM, 1), lambda i: (i, 0)),
        pl.BlockSpec((BM, 1), lambda i: (i, 0)),
        pl.BlockSpec((BM, ADIM), lambda i: (i, 0)),
        pl.BlockSpec((CH, HID2), lambda i: (0, 0)),
        pl.BlockSpec((CH, HID2), lambda i: (0, 0)),
        pl.BlockSpec((ADIM, HID2), lambda i: (0, 0)),
        pl.BlockSpec((1, HID2), lambda i: (0, 0)),
        pl.BlockSpec((HID2, G1), lambda i: (0, 0)),
        pl.BlockSpec((1, G1), lambda i: (0, 0)),
        pl.BlockSpec((G1, 1), lambda i: (0, 0)),
        pl.BlockSpec((1, 1), lambda i: (0, 0)),
    ],
    out_specs=[
        pl.BlockSpec((1, HID2), lambda i: (0, 0)),
        pl.BlockSpec((1, 1), lambda i: (0, 0)),
    ],
    out_shape=[
        jax.ShapeDtypeStruct((1, HID2), jnp.float32),
        jax.ShapeDtypeStruct((1, 1), jnp.float32),
    ],
)


def kernel(n_delay, n_feat, edge_index, action, fc1_W, fc1_b, bn_gamma,
           bn_beta, conv1_W, conv1_b, fc2_W, fc2_b, conv2_W, conv2_b,
           fc3_W, fc3_b):
    ei = edge_index.astype(jnp.int32).reshape(2 * E)

    degs = _degrees(ei)
    out_norm = lax.rsqrt(jnp.clip(degs[0, :N], 1.0, None))
    in_norm_pad = lax.rsqrt(jnp.clip(degs[1], 1.0, None))

    w1t = fc1_W.T
    h1, s1, s2 = _fc1_call(n_delay, n_feat, w1t[:64], w1t[64:],
                           fc1_b.reshape(1, HID1))
    mean = s1 * (1.0 / N)
    var = s2 * (1.0 / N) - mean * mean
    a = bn_gamma.reshape(1, HID1) * lax.rsqrt(var + 1e-5)
    csh = bn_beta.reshape(1, HID1) - mean * a

    w1p = jnp.concatenate(
        [conv1_W, jnp.zeros((HID1, 2 * CH - G1), jnp.float32)], axis=1)
    hw = _hw_call(h1, a, csh, out_norm.reshape(N, 1), w1p)

    zeros2d = jnp.zeros((NPAD, CH), jnp.float32)
    agg = _conv_scatter(ei, hw, zeros2d)
    w01 = _w_scatter(ei, in_norm_pad)

    w2t = fc2_W.T
    wg = w2t[:G1]
    wg32 = jnp.concatenate(
        [wg, jnp.zeros((2 * CH - G1, HID2), jnp.float32)], axis=0)
    b2p = (fc2_b + conv1_b @ wg).reshape(1, HID2)
    _, q = _final_call(
        agg, w01[:, :N].T, in_norm_pad[:N].reshape(N, 1), out_norm.reshape(N, 1),
        action, wg32[:CH], wg32[CH:], w2t[G1:], b2p,
        conv2_W, conv2_b.reshape(1, G1), fc3_W.T, fc3_b.reshape(1, 1))
    return q.reshape(())
